# Initial kernel scaffold; baseline (speedup 1.0000x reference)
import jax
import jax.numpy as jnp
from jax.experimental import pallas as pl


def kernel(score, target):
    raise NotImplementedError("write your pallas kernel here")


# speedup vs baseline: 7.4728x; 7.4728x over previous
import functools

import jax
import jax.numpy as jnp
import numpy as np
from jax.experimental import pallas as pl
from jax.experimental.pallas import tpu as pltpu

THRESH = np.float32(0.7)
C = 19
CHUNK = 32768


def _stage1_body(score_ref, target_ref, pred_ref, loss_ref):
    s = score_ref[0]
    t = target_ref[0]
    m = jnp.max(s, axis=0, keepdims=True)
    e = jnp.exp(s - m)
    se = jnp.sum(e, axis=0, keepdims=True)
    cidx = jax.lax.broadcasted_iota(jnp.int32, (C, CHUNK), 0)
    onehot = cidx == t
    st = jnp.sum(jnp.where(onehot, s, 0.0), axis=0, keepdims=True)
    eg = jnp.sum(jnp.where(onehot, e, 0.0), axis=0, keepdims=True)
    pred_ref[0] = eg / se
    loss_ref[0] = -st


def _stage2_body(k, pred_ref, loss_ref, out_ref):
    x = pred_ref[...]
    c07 = jnp.sum((x <= THRESH).astype(jnp.int32))

    def fast(_):
        return THRESH

    def slow(_):
        xb = jax.lax.bitcast_convert_type(x, jnp.int32)

        def body(_, carry):
            lo, hi = carry
            mid = (lo + hi) // 2
            c = jnp.sum((xb <= mid).astype(jnp.int32))
            take_hi = c >= k + 1
            return (jnp.where(take_hi, lo, mid), jnp.where(take_hi, mid, hi))

        _, hi = jax.lax.fori_loop(
            0, 31, body, (jnp.int32(-1), jnp.int32(0x3F800000)))
        v = jax.lax.bitcast_convert_type(hi, jnp.float32)
        return jnp.maximum(v, THRESH)

    thr = jax.lax.cond(c07 >= k + 1, fast, slow, None)
    keep = (x < thr).astype(jnp.float32)
    ks = jnp.sum(loss_ref[...] * keep)
    kc = jnp.sum(keep)
    out_ref[0, 0] = ks / jnp.maximum(kc, 1.0)


@jax.jit
def kernel(score, target):
    B, Cc, H, W = score.shape
    P = H * W
    n_chunks = P // CHUNK
    steps = B * n_chunks
    k = int(0.7 * H * W)

    score3 = score.reshape(B, Cc, P)
    target3 = target.reshape(B, 1, P)

    pred, loss = pl.pallas_call(
        _stage1_body,
        grid=(steps,),
        in_specs=[
            pl.BlockSpec((1, Cc, CHUNK), lambda i: (i // n_chunks, 0, i % n_chunks)),
            pl.BlockSpec((1, 1, CHUNK), lambda i: (i // n_chunks, 0, i % n_chunks)),
        ],
        out_specs=[
            pl.BlockSpec((1, 1, CHUNK), lambda i: (i, 0, 0)),
            pl.BlockSpec((1, 1, CHUNK), lambda i: (i, 0, 0)),
        ],
        out_shape=[
            jax.ShapeDtypeStruct((steps, 1, CHUNK), jnp.float32),
            jax.ShapeDtypeStruct((steps, 1, CHUNK), jnp.float32),
        ],
        compiler_params=pltpu.CompilerParams(
            dimension_semantics=("arbitrary",),
        ),
    )(score3, target3)

    rows = steps * CHUNK // 4096
    pred2 = pred.reshape(rows, 4096)
    loss2 = loss.reshape(rows, 4096)

    out = pl.pallas_call(
        functools.partial(_stage2_body, k),
        in_specs=[
            pl.BlockSpec(memory_space=pltpu.VMEM),
            pl.BlockSpec(memory_space=pltpu.VMEM),
        ],
        out_specs=pl.BlockSpec(memory_space=pltpu.SMEM),
        out_shape=jax.ShapeDtypeStruct((1, 1), jnp.float32),
    )(pred2, loss2)
    return out[0, 0]


# speedup vs baseline: 11.5655x; 1.5477x over previous
import functools

import jax
import jax.numpy as jnp
import numpy as np
from jax.experimental import pallas as pl
from jax.experimental.pallas import tpu as pltpu

THRESH = np.float32(0.7)
C = 19
CHUNK = 32768


def _stage1_body(score_ref, target_ref, pred_ref, loss_ref):
    t = target_ref[0]
    m = score_ref[0, 0]
    for c in range(1, C):
        m = jnp.maximum(m, score_ref[0, c])
    se = jnp.zeros_like(m)
    st = jnp.zeros_like(m)
    for c in range(C):
        s = score_ref[0, c]
        se = se + jnp.exp(s - m)
        st = jnp.where(t == c, s, st)
    pred_ref[0] = jnp.exp(st - m) / se
    loss_ref[0] = -st


def _stage2_body(k, pred_ref, loss_ref, out_ref):
    x = pred_ref[...]
    c07 = jnp.sum((x <= THRESH).astype(jnp.int32))

    def fast(_):
        return THRESH

    def slow(_):
        xb = jax.lax.bitcast_convert_type(x, jnp.int32)

        def body(_, carry):
            lo, hi = carry
            mid = (lo + hi) // 2
            c = jnp.sum((xb <= mid).astype(jnp.int32))
            take_hi = c >= k + 1
            return (jnp.where(take_hi, lo, mid), jnp.where(take_hi, mid, hi))

        _, hi = jax.lax.fori_loop(
            0, 31, body, (jnp.int32(-1), jnp.int32(0x3F800000)))
        v = jax.lax.bitcast_convert_type(hi, jnp.float32)
        return jnp.maximum(v, THRESH)

    thr = jax.lax.cond(c07 >= k + 1, fast, slow, None)
    keep = (x < thr).astype(jnp.float32)
    ks = jnp.sum(loss_ref[...] * keep)
    kc = jnp.sum(keep)
    out_ref[0, 0] = ks / jnp.maximum(kc, 1.0)


@jax.jit
def kernel(score, target):
    B, Cc, H, W = score.shape
    P = H * W
    LANE = 4096
    SUB = CHUNK // LANE
    n_chunks = P // CHUNK
    steps = B * n_chunks
    rows_per_b = P // LANE
    k = int(0.7 * H * W)

    score4 = score.reshape(B, Cc, rows_per_b, LANE)
    target4 = target.reshape(B, rows_per_b, LANE)

    pred, loss = pl.pallas_call(
        _stage1_body,
        grid=(steps,),
        in_specs=[
            pl.BlockSpec((1, Cc, SUB, LANE),
                         lambda i: (i // n_chunks, 0, i % n_chunks, 0)),
            pl.BlockSpec((1, SUB, LANE),
                         lambda i: (i // n_chunks, i % n_chunks, 0)),
        ],
        out_specs=[
            pl.BlockSpec((1, SUB, LANE), lambda i: (i, 0, 0)),
            pl.BlockSpec((1, SUB, LANE), lambda i: (i, 0, 0)),
        ],
        out_shape=[
            jax.ShapeDtypeStruct((steps, SUB, LANE), jnp.float32),
            jax.ShapeDtypeStruct((steps, SUB, LANE), jnp.float32),
        ],
        compiler_params=pltpu.CompilerParams(
            dimension_semantics=("arbitrary",),
        ),
    )(score4, target4)

    rows = steps * SUB
    pred2 = pred.reshape(rows, LANE)
    loss2 = loss.reshape(rows, LANE)

    out = pl.pallas_call(
        functools.partial(_stage2_body, k),
        in_specs=[
            pl.BlockSpec(memory_space=pltpu.VMEM),
            pl.BlockSpec(memory_space=pltpu.VMEM),
        ],
        out_specs=pl.BlockSpec(memory_space=pltpu.SMEM),
        out_shape=jax.ShapeDtypeStruct((1, 1), jnp.float32),
    )(pred2, loss2)
    return out[0, 0]


# speedup vs baseline: 11.9294x; 1.0315x over previous
import jax
import jax.numpy as jnp
import numpy as np
from jax.experimental import pallas as pl
from jax.experimental.pallas import tpu as pltpu

THRESH = np.float32(0.7)
C = 19
LANE = 4096
SUB = 8
CHUNK = SUB * LANE


def _body(k, steps, score_ref, target_ref, out_ref, pred_buf, loss_buf):
    i = pl.program_id(0)

    @pl.when(i < steps)
    def compute_chunk():
        t = target_ref[0]
        m = score_ref[0, 0]
        for c in range(1, C):
            m = jnp.maximum(m, score_ref[0, c])
        se = jnp.zeros_like(m)
        st = jnp.zeros_like(m)
        for c in range(C):
            s = score_ref[0, c]
            se = se + jnp.exp(s - m)
            st = jnp.where(t == c, s, st)
        rows = pl.ds(i * SUB, SUB)
        pred_buf[rows, :] = jnp.exp(st - m) / se
        loss_buf[rows, :] = -st

    @pl.when(i == steps)
    def epilogue():
        x = pred_buf[...]
        c07 = jnp.sum((x <= THRESH).astype(jnp.int32))

        def fast(_):
            return THRESH

        def slow(_):
            xb = jax.lax.bitcast_convert_type(x, jnp.int32)

            def bisect(_, carry):
                lo, hi = carry
                mid = (lo + hi) // 2
                c = jnp.sum((xb <= mid).astype(jnp.int32))
                take_hi = c >= k + 1
                return (jnp.where(take_hi, lo, mid),
                        jnp.where(take_hi, mid, hi))

            _, hi = jax.lax.fori_loop(
                0, 31, bisect, (jnp.int32(-1), jnp.int32(0x3F800000)))
            v = jax.lax.bitcast_convert_type(hi, jnp.float32)
            return jnp.maximum(v, THRESH)

        thr = jax.lax.cond(c07 >= k + 1, fast, slow, None)
        keep = (x < thr).astype(jnp.float32)
        ks = jnp.sum(loss_buf[...] * keep)
        kc = jnp.sum(keep)
        out_ref[0, 0] = ks / jnp.maximum(kc, 1.0)


@jax.jit
def kernel(score, target):
    B, Cc, H, W = score.shape
    P = H * W
    n_chunks = P // CHUNK
    steps = B * n_chunks
    rows_per_b = P // LANE
    k = int(0.7 * H * W)

    score4 = score.reshape(B, Cc, rows_per_b, LANE)
    target4 = target.reshape(B, rows_per_b, LANE)

    def score_map(i):
        j = jnp.minimum(i, steps - 1)
        return (j // n_chunks, 0, j % n_chunks, 0)

    def target_map(i):
        j = jnp.minimum(i, steps - 1)
        return (j // n_chunks, j % n_chunks, 0)

    body = lambda *refs: _body(k, steps, *refs)

    out = pl.pallas_call(
        body,
        grid=(steps + 1,),
        in_specs=[
            pl.BlockSpec((1, Cc, SUB, LANE), score_map),
            pl.BlockSpec((1, SUB, LANE), target_map),
        ],
        out_specs=pl.BlockSpec(memory_space=pltpu.SMEM),
        out_shape=jax.ShapeDtypeStruct((1, 1), jnp.float32),
        scratch_shapes=[
            pltpu.VMEM((steps * SUB, LANE), jnp.float32),
            pltpu.VMEM((steps * SUB, LANE), jnp.float32),
        ],
        compiler_params=pltpu.CompilerParams(
            dimension_semantics=("arbitrary",),
        ),
    )(score4, target4)
    return out[0, 0]


# speedup vs baseline: 12.7316x; 1.0673x over previous
import jax
import jax.numpy as jnp
import numpy as np
from jax.experimental import pallas as pl
from jax.experimental.pallas import tpu as pltpu

THRESH = np.float32(0.7)
C = 19
LANE = 4096
SUB = 16
CHUNK = SUB * LANE


def _body(k, steps, score_ref, target_ref, out_ref, pred_buf, loss_buf):
    i = pl.program_id(0)

    @pl.when(i < steps)
    def compute_chunk():
        t = target_ref[0]
        m = score_ref[0, 0]
        for c in range(1, C):
            m = jnp.maximum(m, score_ref[0, c])
        se = jnp.zeros_like(m)
        st = jnp.zeros_like(m)
        for c in range(C):
            s = score_ref[0, c]
            se = se + jnp.exp(s - m)
            st = jnp.where(t == c, s, st)
        rows = pl.ds(i * SUB, SUB)
        pred_buf[rows, :] = jnp.exp(st - m) / se
        loss_buf[rows, :] = -st

    @pl.when(i == steps)
    def epilogue():
        x = pred_buf[...]
        c07 = jnp.sum((x <= THRESH).astype(jnp.int32))

        def fast(_):
            return THRESH

        def slow(_):
            xb = jax.lax.bitcast_convert_type(x, jnp.int32)

            def bisect(_, carry):
                lo, hi = carry
                mid = (lo + hi) // 2
                c = jnp.sum((xb <= mid).astype(jnp.int32))
                take_hi = c >= k + 1
                return (jnp.where(take_hi, lo, mid),
                        jnp.where(take_hi, mid, hi))

            _, hi = jax.lax.fori_loop(
                0, 31, bisect, (jnp.int32(-1), jnp.int32(0x3F800000)))
            v = jax.lax.bitcast_convert_type(hi, jnp.float32)
            return jnp.maximum(v, THRESH)

        thr = jax.lax.cond(c07 >= k + 1, fast, slow, None)
        keep = (x < thr).astype(jnp.float32)
        ks = jnp.sum(loss_buf[...] * keep)
        kc = jnp.sum(keep)
        out_ref[0, 0] = ks / jnp.maximum(kc, 1.0)


@jax.jit
def kernel(score, target):
    B, Cc, H, W = score.shape
    P = H * W
    n_chunks = P // CHUNK
    steps = B * n_chunks
    rows_per_b = P // LANE
    k = int(0.7 * H * W)

    score4 = score.reshape(B, Cc, rows_per_b, LANE)
    target4 = target.reshape(B, rows_per_b, LANE)

    def score_map(i):
        j = jnp.minimum(i, steps - 1)
        return (j // n_chunks, 0, j % n_chunks, 0)

    def target_map(i):
        j = jnp.minimum(i, steps - 1)
        return (j // n_chunks, j % n_chunks, 0)

    body = lambda *refs: _body(k, steps, *refs)

    out = pl.pallas_call(
        body,
        grid=(steps + 1,),
        in_specs=[
            pl.BlockSpec((1, Cc, SUB, LANE), score_map),
            pl.BlockSpec((1, SUB, LANE), target_map),
        ],
        out_specs=pl.BlockSpec(memory_space=pltpu.SMEM),
        out_shape=jax.ShapeDtypeStruct((1, 1), jnp.float32),
        scratch_shapes=[
            pltpu.VMEM((steps * SUB, LANE), jnp.float32),
            pltpu.VMEM((steps * SUB, LANE), jnp.float32),
        ],
        compiler_params=pltpu.CompilerParams(
            dimension_semantics=("arbitrary",),
        ),
    )(score4, target4)
    return out[0, 0]


# speedup vs baseline: 13.0422x; 1.0244x over previous
import jax
import jax.numpy as jnp
import numpy as np
from jax.experimental import pallas as pl
from jax.experimental.pallas import tpu as pltpu

THRESH = np.float32(0.7)
C = 19
LANE = 4096
SUB = 32
CHUNK = SUB * LANE


def _body(k, steps, score_ref, target_ref, out_ref, pred_buf, loss_buf):
    i = pl.program_id(0)

    @pl.when(i < steps)
    def compute_chunk():
        t = target_ref[0]
        m = score_ref[0, 0]
        for c in range(1, C):
            m = jnp.maximum(m, score_ref[0, c])
        se = jnp.zeros_like(m)
        st = jnp.zeros_like(m)
        for c in range(C):
            s = score_ref[0, c]
            se = se + jnp.exp(s - m)
            st = jnp.where(t == c, s, st)
        rows = pl.ds(i * SUB, SUB)
        pred_buf[rows, :] = jnp.exp(st - m) / se
        loss_buf[rows, :] = -st

    @pl.when(i == steps)
    def epilogue():
        x = pred_buf[...]
        c07 = jnp.sum((x <= THRESH).astype(jnp.int32))

        def fast(_):
            return THRESH

        def slow(_):
            xb = jax.lax.bitcast_convert_type(x, jnp.int32)

            def bisect(_, carry):
                lo, hi = carry
                mid = (lo + hi) // 2
                c = jnp.sum((xb <= mid).astype(jnp.int32))
                take_hi = c >= k + 1
                return (jnp.where(take_hi, lo, mid),
                        jnp.where(take_hi, mid, hi))

            _, hi = jax.lax.fori_loop(
                0, 31, bisect, (jnp.int32(-1), jnp.int32(0x3F800000)))
            v = jax.lax.bitcast_convert_type(hi, jnp.float32)
            return jnp.maximum(v, THRESH)

        thr = jax.lax.cond(c07 >= k + 1, fast, slow, None)
        keep = (x < thr).astype(jnp.float32)
        ks = jnp.sum(loss_buf[...] * keep)
        kc = jnp.sum(keep)
        out_ref[0, 0] = ks / jnp.maximum(kc, 1.0)


@jax.jit
def kernel(score, target):
    B, Cc, H, W = score.shape
    P = H * W
    n_chunks = P // CHUNK
    steps = B * n_chunks
    rows_per_b = P // LANE
    k = int(0.7 * H * W)

    score4 = score.reshape(B, Cc, rows_per_b, LANE)
    target4 = target.reshape(B, rows_per_b, LANE)

    def score_map(i):
        j = jnp.minimum(i, steps - 1)
        return (j // n_chunks, 0, j % n_chunks, 0)

    def target_map(i):
        j = jnp.minimum(i, steps - 1)
        return (j // n_chunks, j % n_chunks, 0)

    body = lambda *refs: _body(k, steps, *refs)

    out = pl.pallas_call(
        body,
        grid=(steps + 1,),
        in_specs=[
            pl.BlockSpec((1, Cc, SUB, LANE), score_map),
            pl.BlockSpec((1, SUB, LANE), target_map),
        ],
        out_specs=pl.BlockSpec(memory_space=pltpu.SMEM),
        out_shape=jax.ShapeDtypeStruct((1, 1), jnp.float32),
        scratch_shapes=[
            pltpu.VMEM((steps * SUB, LANE), jnp.float32),
            pltpu.VMEM((steps * SUB, LANE), jnp.float32),
        ],
        compiler_params=pltpu.CompilerParams(
            dimension_semantics=("arbitrary",),
        ),
    )(score4, target4)
    return out[0, 0]


# speedup vs baseline: 52.8367x; 4.0512x over previous
import jax
import jax.numpy as jnp
import numpy as np
from jax.experimental import pallas as pl
from jax.experimental.pallas import tpu as pltpu

THRESH = np.float32(0.7)
C = 19
ROWS = 256


def _body(k, steps, score_ref, target_ref, out_ref, pred_buf, loss_buf):
    i = pl.program_id(0)

    @pl.when(i < steps)
    def compute_chunk():
        t = target_ref[0]
        m = score_ref[0, 0]
        for c in range(1, C):
            m = jnp.maximum(m, score_ref[0, c])
        se = jnp.zeros_like(m)
        st = jnp.zeros_like(m)
        for c in range(C):
            s = score_ref[0, c]
            se = se + jnp.exp(s - m)
            st = jnp.where(t == c, s, st)
        rows = pl.ds(i * ROWS, ROWS)
        pred_buf[rows, :] = jnp.exp(st - m) / se
        loss_buf[rows, :] = -st

    @pl.when(i == steps)
    def epilogue():
        x = pred_buf[...]
        c07 = jnp.sum((x <= THRESH).astype(jnp.int32))

        def fast(_):
            return THRESH

        def slow(_):
            xb = jax.lax.bitcast_convert_type(x, jnp.int32)

            def bisect(_, carry):
                lo, hi = carry
                mid = (lo + hi) // 2
                c = jnp.sum((xb <= mid).astype(jnp.int32))
                take_hi = c >= k + 1
                return (jnp.where(take_hi, lo, mid),
                        jnp.where(take_hi, mid, hi))

            _, hi = jax.lax.fori_loop(
                0, 31, bisect, (jnp.int32(-1), jnp.int32(0x3F800000)))
            v = jax.lax.bitcast_convert_type(hi, jnp.float32)
            return jnp.maximum(v, THRESH)

        thr = jax.lax.cond(c07 >= k + 1, fast, slow, None)
        keep = (x < thr).astype(jnp.float32)
        ks = jnp.sum(loss_buf[...] * keep)
        kc = jnp.sum(keep)
        out_ref[0, 0] = ks / jnp.maximum(kc, 1.0)


@jax.jit
def kernel(score, target):
    B, Cc, H, W = score.shape
    n_chunks = H // ROWS
    steps = B * n_chunks
    k = int(0.7 * H * W)

    def score_map(i):
        j = jnp.minimum(i, steps - 1)
        return (j // n_chunks, 0, j % n_chunks, 0)

    def target_map(i):
        j = jnp.minimum(i, steps - 1)
        return (j // n_chunks, j % n_chunks, 0)

    body = lambda *refs: _body(k, steps, *refs)

    out = pl.pallas_call(
        body,
        grid=(steps + 1,),
        in_specs=[
            pl.BlockSpec((1, Cc, ROWS, W), score_map),
            pl.BlockSpec((1, ROWS, W), target_map),
        ],
        out_specs=pl.BlockSpec(memory_space=pltpu.SMEM),
        out_shape=jax.ShapeDtypeStruct((1, 1), jnp.float32),
        scratch_shapes=[
            pltpu.VMEM((steps * ROWS, W), jnp.float32),
            pltpu.VMEM((steps * ROWS, W), jnp.float32),
        ],
        compiler_params=pltpu.CompilerParams(
            dimension_semantics=("arbitrary",),
        ),
    )(score, target)
    return out[0, 0]
